# baseline (speedup 1.0000x reference)
import jax
import jax.numpy as jnp
from jax.experimental import pallas as pl


_N = 4608
_Q = 512
_E = 32
_R = 256
_K = 3
_IMAX = 0x7FFFFFFF


def _exact_dist(wrow, xb):
    d = wrow - xb
    sq = d * d
    blocks = []
    for g in range(4):
        b = sq[:, 8 * g:8 * g + 8]
        u = b[:, 0:4] + b[:, 4:8]
        v = u[:, 0:2] + u[:, 2:4]
        blocks.append(v[:, 0:1] + v[:, 1:2])
    return ((blocks[0] + blocks[1]) + blocks[2]) + blocks[3]


def _vq_body(x_ref, wt_ref, w_ref, qd_ref, qi_ref):
    xb = x_ref[...]
    wt = wt_ref[...]
    w = w_ref[...]

    wn = jnp.sum(wt * wt, axis=0)[None, :]
    xw2 = jax.lax.dot(xb, wt + wt,
                      precision=jax.lax.Precision.HIGHEST)
    s = wn - xw2

    u = jax.lax.bitcast_convert_type(s, jnp.int32)
    k = u ^ jax.lax.shift_right_logical(
        jax.lax.shift_right_arithmetic(u, 31), 1)
    qiota = jax.lax.broadcasted_iota(jnp.int32, (_R, _Q), 1)
    key = (k & jnp.int32(~511)) | qiota

    best_d = None
    best_i = None
    best_row = None
    for _ in range(_K):
        mk = jnp.min(key, axis=1, keepdims=True)
        hit = key == mk
        key = jnp.where(hit, _IMAX, key)
        idx = mk[:, 0] & 511
        wrow = jax.lax.dot(hit.astype(jnp.float32), w,
                           precision=jax.lax.Precision.HIGHEST)
        d = _exact_dist(wrow, xb)[:, 0]
        if best_d is None:
            best_d, best_i, best_row = d, idx, wrow
        else:
            take = (d < best_d) | ((d == best_d) & (idx < best_i))
            best_d = jnp.where(take, d, best_d)
            best_i = jnp.where(take, idx, best_i)
            best_row = jnp.where(take[:, None], wrow, best_row)

    qd_ref[...] = best_row
    qi_ref[0, 0, :] = best_i


@jax.jit
def _vq(xf, wt, w):
    nb = _N // _R
    qd, qi = pl.pallas_call(
        _vq_body,
        grid=(nb,),
        in_specs=[
            pl.BlockSpec((_R, _E), lambda i: (i, 0)),
            pl.BlockSpec((_E, _Q), lambda i: (0, 0)),
            pl.BlockSpec((_Q, _E), lambda i: (0, 0)),
        ],
        out_specs=[
            pl.BlockSpec((_R, _E), lambda i: (i, 0)),
            pl.BlockSpec((1, 1, _R), lambda i: (i, 0, 0)),
        ],
        out_shape=[
            jax.ShapeDtypeStruct((_N, _E), jnp.float32),
            jax.ShapeDtypeStruct((nb, 1, _R), jnp.int32),
        ],
    )(xf, wt, w)
    return qd, qi


def kernel(x, weight):
    xf = x.reshape(_N, _E)
    qd, qi = _vq(xf, weight.T, weight)
    return qd.reshape(x.shape), qi.reshape(x.shape[:-1])
